# baseline (speedup 1.0000x reference)
import functools
import math

import jax
import jax.numpy as jnp
from jax import lax
from jax.experimental import pallas as pl
from jax.experimental.pallas import tpu as pltpu
from jax.experimental.pallas import tpu_sc as plsc

_VOCAB = 100000
_D = 1024
_B = 4
_L = 2048
_NFLAT = _B * _L
_NC = 2
_NS = 16
_NW = _NC * _NS
_ROWS_PER_W = _NFLAT // _NW
_CH = 16
_NCHUNK = _ROWS_PER_W // _CH
_SPANS_PER_BATCH = _L // _ROWS_PER_W
_SCALE = math.sqrt(_D)


def _pe_body(o_ref):
    i = pl.program_id(0)
    rows = o_ref.shape[0]
    half = _D // 2
    pos = (lax.broadcasted_iota(jnp.int32, (rows, half), 0) + i * rows).astype(
        jnp.float32)
    col = lax.broadcasted_iota(jnp.int32, (rows, half), 1)
    dd = col.astype(jnp.float32) * (1.0 / half)
    rate = jnp.exp(dd * (-math.log(10000.0)))
    ang = pos * rate
    o_ref[:, :half] = jnp.sin(ang)
    o_ref[:, half:] = jnp.cos(ang)


def _make_pe():
    blk = 256
    return pl.pallas_call(
        _pe_body,
        grid=(_L // blk,),
        out_specs=pl.BlockSpec((blk, _D), lambda i: (i, 0)),
        out_shape=jax.ShapeDtypeStruct((_L, _D), jnp.float32),
    )()


def _sc_body(table_h, idx_h, pe_h, out_h, idx_v, g_v, p_v, sem):
    w = lax.axis_index("s") * _NC + lax.axis_index("c")
    row_base = w * _ROWS_PER_W
    pos_base = lax.rem(w, _SPANS_PER_BATCH) * _ROWS_PER_W
    pltpu.sync_copy(idx_h.at[w], idx_v)

    def chunk(c, carry):
        pltpu.async_copy(table_h.at[idx_v.at[c]], g_v, sem).wait()
        pltpu.sync_copy(pe_h.at[pl.ds(pos_base + c * _CH, _CH)], p_v)

        def fma(i, carry2):
            r = i // (_D // 16)
            j = lax.rem(i, _D // 16) * 16
            g = g_v[r, pl.ds(j, 16)]
            p = p_v[r, pl.ds(j, 16)]
            g_v[r, pl.ds(j, 16)] = g * _SCALE + p
            return carry2

        lax.fori_loop(0, _CH * (_D // 16), fma, 0)
        pltpu.sync_copy(g_v, out_h.at[pl.ds(row_base + c * _CH, _CH)])
        return carry

    lax.fori_loop(0, _NCHUNK, chunk, 0)


@functools.partial(
    pl.kernel,
    mesh=plsc.VectorSubcoreMesh(core_axis_name="c", subcore_axis_name="s"),
    out_type=jax.ShapeDtypeStruct((_NFLAT, _D), jnp.float32),
    scratch_types=[
        pltpu.VMEM((_NCHUNK, _CH), jnp.int32),
        pltpu.VMEM((_CH, _D), jnp.float32),
        pltpu.VMEM((_CH, _D), jnp.float32),
        pltpu.SemaphoreType.DMA,
    ],
)
def _sc_embed(table_h, idx_h, pe_h, out_h, idx_v, g_v, p_v, sem):
    _sc_body(table_h, idx_h, pe_h, out_h, idx_v, g_v, p_v, sem)


def kernel(x, table):
    pe = _make_pe()
    idx = x.astype(jnp.int32).reshape(_NW, _NCHUNK, _CH)
    out = _sc_embed(table, idx, pe)
    return out.reshape(_B, _L, _D)


# speedup vs baseline: 2.7498x; 2.7498x over previous
import functools
import math

import jax
import jax.numpy as jnp
from jax import lax
from jax.experimental import pallas as pl
from jax.experimental.pallas import tpu as pltpu
from jax.experimental.pallas import tpu_sc as plsc

_VOCAB = 100000
_D = 1024
_B = 4
_L = 2048
_NFLAT = _B * _L
_NC = 2
_NS = 16
_NW = _NC * _NS
_POS_PER_W = _L // _NW
_CPOS = 8
_NCHUNK = _POS_PER_W // _CPOS
_CROWS = _CPOS * _B
_STAGE_POS = 32
_SCALE = math.sqrt(_D)
_JV = _D // 16


def _pe_body(o_ref):
    i = pl.program_id(0)
    rows = o_ref.shape[0]
    half = _D // 2
    pos = (lax.broadcasted_iota(jnp.int32, (rows, half), 0) + i * rows).astype(
        jnp.float32)
    col = lax.broadcasted_iota(jnp.int32, (rows, half), 1)
    dd = col.astype(jnp.float32) * (1.0 / half)
    rate = jnp.exp(dd * (-math.log(10000.0)))
    ang = pos * rate
    o_ref[:, :half] = jnp.sin(ang)
    o_ref[:, half:] = jnp.cos(ang)


def _make_pe():
    blk = 256
    return pl.pallas_call(
        _pe_body,
        grid=(_L // blk,),
        out_specs=pl.BlockSpec((blk, _D), lambda i: (i, 0)),
        out_shape=jax.ShapeDtypeStruct((_L, _D), jnp.float32),
    )()


def _sc_body(table_h, idx_h, pe_h, out_h, idx_v, pe_v, g0, g1,
             sem_g0, sem_g1, sem_o0, sem_o1, sem_pe):
    w = lax.axis_index("s") * _NC + lax.axis_index("c")
    pos0 = w * _POS_PER_W
    pltpu.sync_copy(idx_h.at[w], idx_v)

    g_set = (g0, g1)
    sem_g = (sem_g0, sem_g1)
    sem_o = (sem_o0, sem_o1)
    out_handles = [None, None]
    gather_handles = [None, None]

    def issue_gather(c):
        s = c % 2
        gather_handles[s] = pltpu.async_copy(
            table_h.at[idx_v.at[c]], g_set[s], sem_g[s])

    pltpu.sync_copy(pe_h.at[pl.ds(pos0, _STAGE_POS)], pe_v)
    issue_gather(0)
    pe1_handle = None

    for c in range(_NCHUNK):
        s = c % 2
        g_v = g_set[s]
        gather_handles[s].wait()
        if c + 1 < _NCHUNK:
            ns = (c + 1) % 2
            if out_handles[ns] is not None:
                for h in out_handles[ns]:
                    h.wait()
                out_handles[ns] = None
            issue_gather(c + 1)
        if c == _NCHUNK // 2:
            pe1_handle.wait()
        pe_base = (c % (_STAGE_POS // _CPOS)) * _CPOS

        @plsc.parallel_loop(0, _CPOS * _JV, unroll=4)
        def _fma(i):
            p = lax.shift_right_logical(i, 6)
            j = pl.multiple_of(
                lax.shift_left(jnp.bitwise_and(i, _JV - 1), 4), 16)
            pvec = pe_v[pe_base + p, pl.ds(j, 16)]
            for b in range(_B):
                r = b * _CPOS + p
                g = g_v[r, pl.ds(j, 16)]
                g_v[r, pl.ds(j, 16)] = g * _SCALE + pvec

        if c == _NCHUNK // 2 - 1:
            pe1_handle = pltpu.async_copy(
                pe_h.at[pl.ds(pos0 + _STAGE_POS, _STAGE_POS)], pe_v, sem_pe)
        hs = []
        for b in range(_B):
            hs.append(pltpu.async_copy(
                g_v.at[pl.ds(b * _CPOS, _CPOS)],
                out_h.at[pl.ds(b * _L + pos0 + c * _CPOS, _CPOS)],
                sem_o[s]))
        out_handles[s] = hs

    for hlist in out_handles:
        if hlist is not None:
            for h in hlist:
                h.wait()


@functools.partial(
    pl.kernel,
    mesh=plsc.VectorSubcoreMesh(core_axis_name="c", subcore_axis_name="s"),
    out_type=jax.ShapeDtypeStruct((_NFLAT, _D), jnp.float32),
    scratch_types=[
        pltpu.VMEM((_NCHUNK, _CROWS), jnp.int32),
        pltpu.VMEM((_STAGE_POS, _D), jnp.float32),
        pltpu.VMEM((_CROWS, _D), jnp.float32),
        pltpu.VMEM((_CROWS, _D), jnp.float32),
        pltpu.SemaphoreType.DMA,
        pltpu.SemaphoreType.DMA,
        pltpu.SemaphoreType.DMA,
        pltpu.SemaphoreType.DMA,
        pltpu.SemaphoreType.DMA,
    ],
)
def _sc_embed(table_h, idx_h, pe_h, out_h, idx_v, pe_v, g0, g1,
              sem_g0, sem_g1, sem_o0, sem_o1, sem_pe):
    _sc_body(table_h, idx_h, pe_h, out_h, idx_v, pe_v, g0, g1,
             sem_g0, sem_g1, sem_o0, sem_o1, sem_pe)


def kernel(x, table):
    pe = _make_pe()
    idx = (x.astype(jnp.int32)
           .reshape(_B, _NW, _NCHUNK, _CPOS)
           .transpose(1, 2, 0, 3)
           .reshape(_NW, _NCHUNK, _CROWS))
    out = _sc_embed(table, idx, pe)
    return out.reshape(_B, _L, _D)


# speedup vs baseline: 3.2526x; 1.1828x over previous
import functools
import math

import jax
import jax.numpy as jnp
from jax import lax
from jax.experimental import pallas as pl
from jax.experimental.pallas import tpu as pltpu
from jax.experimental.pallas import tpu_sc as plsc

_VOCAB = 100000
_D = 1024
_B = 4
_L = 2048
_NFLAT = _B * _L
_NC = 2
_NS = 16
_NW = _NC * _NS
_POS_PER_W = _L // _NW
_CPOS = 8
_NCHUNK = _POS_PER_W // _CPOS
_CROWS = _CPOS * _B
_STAGE_POS = 32
_SCALE = math.sqrt(_D)
_JV = _D // 16


_PE_BLK = 256


def _pe_body(o_ref):
    half = _D // 2
    pos = lax.broadcasted_iota(jnp.int32, (_PE_BLK, half), 0).astype(jnp.float32)
    col = lax.broadcasted_iota(jnp.int32, (_PE_BLK, half), 1)
    dd = col.astype(jnp.float32) * (1.0 / half)
    rate = jnp.exp(dd * (-math.log(10000.0)))
    s = jnp.sin(pos * rate)
    c = jnp.cos(pos * rate)
    step_c = jnp.cos(float(_PE_BLK) * rate)
    step_s = jnp.sin(float(_PE_BLK) * rate)
    for k in range(_L // _PE_BLK):
        o_ref[pl.ds(k * _PE_BLK, _PE_BLK), :half] = s
        o_ref[pl.ds(k * _PE_BLK, _PE_BLK), half:] = c
        if k + 1 < _L // _PE_BLK:
            s, c = s * step_c + c * step_s, c * step_c - s * step_s


def _make_pe():
    return pl.pallas_call(
        _pe_body,
        out_shape=jax.ShapeDtypeStruct((_L, _D), jnp.float32),
    )()


def _sc_body(table_h, idx_h, pe_h, out_h, idx_v, pe_v, g0, g1,
             sem_g0, sem_g1, sem_o0, sem_o1, sem_pe):
    w = lax.axis_index("s") * _NC + lax.axis_index("c")
    pos0 = w * _POS_PER_W
    pltpu.sync_copy(idx_h.at[w], idx_v)

    g_set = (g0, g1)
    sem_g = (sem_g0, sem_g1)
    sem_o = (sem_o0, sem_o1)
    out_handles = [None, None]
    gather_handles = [None, None]

    def issue_gather(c):
        s = c % 2
        gather_handles[s] = pltpu.async_copy(
            table_h.at[idx_v.at[c]], g_set[s], sem_g[s])

    pltpu.sync_copy(pe_h.at[pl.ds(pos0, _STAGE_POS)], pe_v)
    issue_gather(0)
    pe1_handle = None

    for c in range(_NCHUNK):
        s = c % 2
        g_v = g_set[s]
        gather_handles[s].wait()
        if c + 1 < _NCHUNK:
            ns = (c + 1) % 2
            if out_handles[ns] is not None:
                for h in out_handles[ns]:
                    h.wait()
                out_handles[ns] = None
            issue_gather(c + 1)
        if c == _NCHUNK // 2:
            pe1_handle.wait()
        pe_base = (c % (_STAGE_POS // _CPOS)) * _CPOS

        @plsc.parallel_loop(0, _CPOS * _JV, unroll=4)
        def _fma(i):
            p = lax.shift_right_logical(i, 6)
            j = pl.multiple_of(
                lax.shift_left(jnp.bitwise_and(i, _JV - 1), 4), 16)
            pvec = pe_v[pe_base + p, pl.ds(j, 16)]
            for b in range(_B):
                r = b * _CPOS + p
                g = g_v[r, pl.ds(j, 16)]
                g_v[r, pl.ds(j, 16)] = g * _SCALE + pvec

        if c == _NCHUNK // 2 - 1:
            pe1_handle = pltpu.async_copy(
                pe_h.at[pl.ds(pos0 + _STAGE_POS, _STAGE_POS)], pe_v, sem_pe)
        hs = []
        for b in range(_B):
            hs.append(pltpu.async_copy(
                g_v.at[pl.ds(b * _CPOS, _CPOS)],
                out_h.at[pl.ds(b * _L + pos0 + c * _CPOS, _CPOS)],
                sem_o[s]))
        out_handles[s] = hs

    for hlist in out_handles:
        if hlist is not None:
            for h in hlist:
                h.wait()


@functools.partial(
    pl.kernel,
    mesh=plsc.VectorSubcoreMesh(core_axis_name="c", subcore_axis_name="s"),
    out_type=jax.ShapeDtypeStruct((_NFLAT, _D), jnp.float32),
    scratch_types=[
        pltpu.VMEM((_NCHUNK, _CROWS), jnp.int32),
        pltpu.VMEM((_STAGE_POS, _D), jnp.float32),
        pltpu.VMEM((_CROWS, _D), jnp.float32),
        pltpu.VMEM((_CROWS, _D), jnp.float32),
        pltpu.SemaphoreType.DMA,
        pltpu.SemaphoreType.DMA,
        pltpu.SemaphoreType.DMA,
        pltpu.SemaphoreType.DMA,
        pltpu.SemaphoreType.DMA,
    ],
)
def _sc_embed(table_h, idx_h, pe_h, out_h, idx_v, pe_v, g0, g1,
              sem_g0, sem_g1, sem_o0, sem_o1, sem_pe):
    _sc_body(table_h, idx_h, pe_h, out_h, idx_v, pe_v, g0, g1,
             sem_g0, sem_g1, sem_o0, sem_o1, sem_pe)


def kernel(x, table):
    pe = _make_pe()
    idx = (x.astype(jnp.int32)
           .reshape(_B, _NW, _NCHUNK, _CPOS)
           .transpose(1, 2, 0, 3)
           .reshape(_NW, _NCHUNK, _CROWS))
    out = _sc_embed(table, idx, pe)
    return out.reshape(_B, _L, _D)
